# Initial kernel scaffold; baseline (speedup 1.0000x reference)
import jax
import jax.numpy as jnp
from jax.experimental import pallas as pl


def kernel(z, mode, embedding):
    raise NotImplementedError("write your pallas kernel here")


# speedup vs baseline: 3.4969x; 3.4969x over previous
import functools

import jax
import jax.numpy as jnp
from jax.experimental import pallas as pl

N_E = 1024
E_DIM = 64
BETA = 0.25
N_TOK = 16 * 32 * 32
BT = 512
NB = N_TOK // BT


def _vq_body(z_ref, emb_ref, enc_ref, zq_ref, idx_ref,
             loss_ref, counts_ref, perp_ref):
    i = pl.program_id(0)
    zb = z_ref[...]
    emb = emb_ref[...]
    z_sq = jnp.sum(zb * zb, axis=1, keepdims=True)
    e_sq = jnp.sum(emb * emb, axis=1)
    prod = jax.lax.dot_general(zb, emb, (((1,), (1,)), ((), ())))
    d = (z_sq + e_sq[None, :]) - 2.0 * prod
    dmin = jnp.min(d, axis=1, keepdims=True)
    lanes = jax.lax.broadcasted_iota(jnp.int32, d.shape, 1)
    idx = jnp.min(jnp.where(d == dmin, lanes, jnp.int32(N_E)), axis=1)
    enc = (lanes == idx[:, None]).astype(jnp.float32)
    enc_ref[...] = enc
    zq = jnp.dot(enc, emb)
    zq_ref[...] = zq
    idx_ref[0, 0, :] = idx

    @pl.when(i == 0)
    def _init():
        loss_ref[...] = jnp.zeros_like(loss_ref)
        counts_ref[...] = jnp.zeros_like(counts_ref)
        perp_ref[...] = jnp.zeros_like(perp_ref)

    loss_ref[...] += jnp.sum((zq - zb) ** 2)[None, None]
    counts_ref[...] += jnp.sum(enc, axis=0, keepdims=True)

    @pl.when(i == NB - 1)
    def _finish():
        total = loss_ref[...]
        loss_ref[...] = total * ((1.0 + BETA) / (N_TOK * E_DIM))
        e_mean = counts_ref[...] / N_TOK
        perp_ref[...] = jnp.exp(
            -jnp.sum(e_mean * jnp.log(e_mean + 1e-10)))[None, None]


@functools.partial(jax.jit, static_argnames=("interpret",))
def _vq(z_flat, embedding, interpret=False):
    out = pl.pallas_call(
        _vq_body,
        grid=(NB,),
        in_specs=[
            pl.BlockSpec((BT, E_DIM), lambda i: (i, 0)),
            pl.BlockSpec((N_E, E_DIM), lambda i: (0, 0)),
        ],
        out_specs=[
            pl.BlockSpec((BT, N_E), lambda i: (i, 0)),
            pl.BlockSpec((BT, E_DIM), lambda i: (i, 0)),
            pl.BlockSpec((1, 1, BT), lambda i: (i, 0, 0)),
            pl.BlockSpec((1, 1), lambda i: (0, 0)),
            pl.BlockSpec((1, N_E), lambda i: (0, 0)),
            pl.BlockSpec((1, 1), lambda i: (0, 0)),
        ],
        out_shape=[
            jax.ShapeDtypeStruct((N_TOK, N_E), jnp.float32),
            jax.ShapeDtypeStruct((N_TOK, E_DIM), jnp.float32),
            jax.ShapeDtypeStruct((NB, 1, BT), jnp.int32),
            jax.ShapeDtypeStruct((1, 1), jnp.float32),
            jax.ShapeDtypeStruct((1, N_E), jnp.float32),
            jax.ShapeDtypeStruct((1, 1), jnp.float32),
        ],
        interpret=interpret,
    )(z_flat, embedding)
    return out


def kernel(z, mode, embedding):
    del mode
    b, c, h, w = z.shape
    z_p = jnp.transpose(z, (0, 2, 3, 1))
    z_flat = z_p.reshape(-1, E_DIM)
    enc, zq, idx3, loss, counts, perp = _vq(z_flat, embedding)
    z_q = jnp.transpose(zq.reshape(b, h, w, c), (0, 3, 1, 2))
    idx_out = idx3.reshape(b, h, w)
    return (loss[0, 0], z_q, perp[0, 0], enc, idx_out)


# speedup vs baseline: 3.5138x; 1.0048x over previous
import functools

import jax
import jax.numpy as jnp
from jax.experimental import pallas as pl
from jax.experimental.pallas import tpu as pltpu

N_E = 1024
E_DIM = 64
BETA = 0.25
N_TOK = 16 * 32 * 32
BT = 512
NB = N_TOK // BT


def _vq_body(z_ref, emb_ref, enc_ref, zq_ref, idx_ref,
             loss_ref, counts_ref, perp_ref, neg2e_ref, esq_ref):
    i = pl.program_id(0)

    @pl.when(i == 0)
    def _prep():
        emb0 = emb_ref[...]
        neg2e_ref[...] = -2.0 * emb0
        esq_ref[...] = jnp.sum(emb0 * emb0, axis=1)[None, :]

    zb = z_ref[...]
    emb = emb_ref[...]
    z_sq = jnp.sum(zb * zb, axis=1, keepdims=True)
    prod = jax.lax.dot_general(zb, neg2e_ref[...], (((1,), (1,)), ((), ())))
    d = (z_sq + esq_ref[...]) + prod
    dmin = jnp.min(d, axis=1, keepdims=True)
    lanes = jax.lax.broadcasted_iota(jnp.int32, d.shape, 1)
    idx = jnp.min(jnp.where(d == dmin, lanes, jnp.int32(N_E)), axis=1)
    enc = (lanes == idx[:, None]).astype(jnp.float32)
    enc_ref[...] = enc
    zq = jnp.dot(enc, emb)
    zq_ref[...] = zq
    idx_ref[0, 0, :] = idx

    @pl.when(i == 0)
    def _init():
        loss_ref[...] = jnp.zeros_like(loss_ref)
        counts_ref[...] = jnp.zeros_like(counts_ref)
        perp_ref[...] = jnp.zeros_like(perp_ref)

    loss_ref[...] += jnp.sum((zq - zb) ** 2)[None, None]
    counts_ref[...] += jnp.sum(enc, axis=0, keepdims=True)

    @pl.when(i == NB - 1)
    def _finish():
        total = loss_ref[...]
        loss_ref[...] = total * ((1.0 + BETA) / (N_TOK * E_DIM))
        e_mean = counts_ref[...] / N_TOK
        perp_ref[...] = jnp.exp(
            -jnp.sum(e_mean * jnp.log(e_mean + 1e-10)))[None, None]


@functools.partial(jax.jit, static_argnames=("interpret",))
def _vq(z_flat, embedding, interpret=False):
    out = pl.pallas_call(
        _vq_body,
        grid=(NB,),
        in_specs=[
            pl.BlockSpec((BT, E_DIM), lambda i: (i, 0)),
            pl.BlockSpec((N_E, E_DIM), lambda i: (0, 0)),
        ],
        out_specs=[
            pl.BlockSpec((BT, N_E), lambda i: (i, 0)),
            pl.BlockSpec((BT, E_DIM), lambda i: (i, 0)),
            pl.BlockSpec((1, 1, BT), lambda i: (i, 0, 0)),
            pl.BlockSpec((1, 1), lambda i: (0, 0)),
            pl.BlockSpec((1, N_E), lambda i: (0, 0)),
            pl.BlockSpec((1, 1), lambda i: (0, 0)),
        ],
        out_shape=[
            jax.ShapeDtypeStruct((N_TOK, N_E), jnp.float32),
            jax.ShapeDtypeStruct((N_TOK, E_DIM), jnp.float32),
            jax.ShapeDtypeStruct((NB, 1, BT), jnp.int32),
            jax.ShapeDtypeStruct((1, 1), jnp.float32),
            jax.ShapeDtypeStruct((1, N_E), jnp.float32),
            jax.ShapeDtypeStruct((1, 1), jnp.float32),
        ],
        scratch_shapes=[
            pltpu.VMEM((N_E, E_DIM), jnp.float32),
            pltpu.VMEM((1, N_E), jnp.float32),
        ],
        interpret=interpret,
    )(z_flat, embedding)
    return out


def kernel(z, mode, embedding):
    del mode
    b, c, h, w = z.shape
    z_p = jnp.transpose(z, (0, 2, 3, 1))
    z_flat = z_p.reshape(-1, E_DIM)
    enc, zq, idx3, loss, counts, perp = _vq(z_flat, embedding)
    z_q = jnp.transpose(zq.reshape(b, h, w, c), (0, 3, 1, 2))
    idx_out = idx3.reshape(b, h, w)
    return (loss[0, 0], z_q, perp[0, 0], enc, idx_out)
